# Initial kernel scaffold; baseline (speedup 1.0000x reference)
import jax
import jax.numpy as jnp
from jax.experimental import pallas as pl


def kernel(x, pe, edge_index, edge_attr, batch, params):
    raise NotImplementedError("write your pallas kernel here")


# speedup vs baseline: 1.5247x; 1.5247x over previous
import functools

import jax
import jax.numpy as jnp
import numpy as np
from jax import lax
from jax.experimental import pallas as pl

_G = 64
_NPG = 32
_N = _G * _NPG
_EPG = 512
_E = _G * _EPG
_FEA = 128
_C = 256
_L = 4
_T = 10
_H = 8
_DH = _C // _H
_PED = 20
_BH = 64
_MINR = 0.1

_NB = 8
_GPB = _G // _NB
_R = _GPB * _NPG
_EPB = _GPB * _EPG
_ECH = 1024
_W = 128

_dot = functools.partial(lax.dot_general, preferred_element_type=jnp.float32,
                         precision=lax.Precision.HIGHEST)
_dot_fast = functools.partial(lax.dot_general, preferred_element_type=jnp.float32)


def _fused_body(x_ref, pe_ref, src_ref, dst_ref,
                embw_ref, embb_ref, pew_ref, peb_ref,
                bw1_ref, bb1_ref, bw2_ref, bb2_ref,
                g1w_ref, g1b_ref, g2w_ref, g2b_ref, eps_ref,
                sw_ref,
                wq_ref, bq_ref, wk_ref, bk_ref, wv_ref, bv_ref,
                wo_ref, bo_ref,
                linw_ref, linb_ref,
                logits_ref, tr_ref, lg_ref):
    i = pl.program_id(0)
    f32 = jnp.float32

    row_id = lax.broadcasted_iota(jnp.int32, (_R, _R), 0)
    col_id = lax.broadcasted_iota(jnp.int32, (_R, _R), 1)
    same = (row_id // _NPG) == (col_id // _NPG)
    same_f = same.astype(f32)
    eye_f = (row_id == col_id).astype(f32)
    ones_rr = jnp.ones((_R, _R), f32)
    p_sum = ((lax.broadcasted_iota(jnp.int32, (_GPB, _R), 1) // _NPG)
             == lax.broadcasted_iota(jnp.int32, (_GPB, _R), 0)).astype(f32)
    lane_w = lax.broadcasted_iota(jnp.int32, (1, _W), 1)

    def _bcast_cols(col):
        return _dot(ones_rr, eye_f * col, (((1,), (0,)), ((), ())))

    h = _dot(x_ref[...], embw_ref[...], (((1,), (0,)), ((), ()))) + embb_ref[...]
    h = h + _dot(pe_ref[...], pew_ref[...], (((1,), (0,)), ((), ()))) + peb_ref[...]

    pooled = _dot(same_f, h, (((1,), (0,)), ((), ()))) * (1.0 / _NPG)
    hid = jnp.maximum(
        _dot(pooled, bw1_ref[...], (((1,), (0,)), ((), ()))) + bb1_ref[...], 0.0)
    bd = _dot(hid, bw2_ref[...], (((1,), (0,)), ((), ()))) + bb2_ref[...]
    trs = _MINR + (1.0 - _MINR) * jax.nn.sigmoid(bd)
    lgs = jax.nn.sigmoid(bd)
    tr_ref[...] = _dot(p_sum, trs, (((1,), (0,)), ((), ()))) * (1.0 / _NPG)
    lg_ref[...] = _dot(p_sum, lgs, (((1,), (0,)), ((), ()))) * (1.0 / _NPG)

    base = i * _R
    srcl = src_ref[0] - base
    dstl = dst_ref[0] - base
    rid = lax.broadcasted_iota(jnp.int32, (_R, _ECH), 0)
    adj = jnp.zeros((_R, _R), f32)
    for c in range(_EPB // _ECH):
        sl = srcl[:, c * _ECH:(c + 1) * _ECH]
        dl = dstl[:, c * _ECH:(c + 1) * _ECH]
        ohd = (dl == rid).astype(jnp.bfloat16)
        ohs = (sl == rid).astype(jnp.bfloat16)
        adj = adj + _dot_fast(ohd, ohs, (((1,), (1,)), ((), ())))

    neg = jnp.float32(-1e30)
    inv_sqrt = jnp.float32(1.0 / np.sqrt(_DH))

    for l in range(_L):
        agg = _dot(adj, h, (((1,), (0,)), ((), ())))
        eps = eps_ref[0:1, l:l + 1]
        z = h + eps * h + agg
        t1 = jnp.maximum(
            _dot(z, g1w_ref[l], (((1,), (0,)), ((), ()))) + g1b_ref[l], 0.0)
        h_local = h + _dot(t1, g2w_ref[l], (((1,), (0,)), ((), ()))) + g2b_ref[l]

        s_col = jnp.sum(h * sw_ref[l], axis=1, keepdims=True)
        smat = _bcast_cols(s_col)
        lt = smat < s_col
        eq = smat == s_col
        rank_col = jnp.sum(
            (same & (lt | (eq & (col_id < row_id)))).astype(f32),
            axis=1, keepdims=True)

        trl = jnp.sum(trs * (lane_w == l).astype(f32), axis=1, keepdims=True)
        lgl = jnp.sum(lgs * (lane_w == (_L + l)).astype(f32), axis=1, keepdims=True)
        idx_col = jnp.clip(jnp.floor((1.0 - trl) * float(_NPG - 1)),
                           0.0, float(_NPG - 1))
        dmat = _bcast_cols(rank_col - idx_col)
        selmat = (dmat == 0.0).astype(f32)
        thr_col = jnp.sum(same_f * smat * selmat, axis=1, keepdims=True)
        m_col = jax.nn.sigmoid(s_col - thr_col)

        bias = jnp.where(same, _bcast_cols(jnp.log(m_col + 1e-6)), neg)
        o_acc = bo_ref[l] + jnp.zeros((_R, _C), f32)
        for hh in range(_H):
            q = _dot(h, wq_ref[l, hh], (((1,), (0,)), ((), ()))) + bq_ref[l, hh]
            k = _dot(h, wk_ref[l, hh], (((1,), (0,)), ((), ()))) + bk_ref[l, hh]
            v = _dot(h, wv_ref[l, hh], (((1,), (0,)), ((), ()))) + bv_ref[l, hh]
            sc = _dot(q, k, (((1,), (1,)), ((), ()))) * inv_sqrt + bias
            p = jax.nn.softmax(sc, axis=-1)
            ov = _dot(p, v, (((1,), (0,)), ((), ())))
            o_acc = o_acc + _dot(ov, wo_ref[l, hh], (((1,), (0,)), ((), ())))
        o = o_acc * m_col
        hsum = h_local + h + lgl * o
        mu = jnp.mean(hsum, axis=-1, keepdims=True)
        d = hsum - mu
        var = jnp.mean(d * d, axis=-1, keepdims=True)
        h = d / jnp.sqrt(var + 1e-5)

    out = _dot(h, linw_ref[...], (((1,), (0,)), ((), ())))
    logits_ref[...] = _dot(p_sum, out, (((1,), (0,)), ((), ()))) + linb_ref[...]


def _full(shape):
    nd = len(shape)
    return pl.BlockSpec(shape, lambda i, _nd=nd: (0,) * _nd)


def _pad_lanes(a, w=_W):
    return jnp.pad(a, ((0, 0),) * (a.ndim - 1) + ((0, w - a.shape[-1]),))


def kernel(x, pe, edge_index, edge_attr, batch, params):
    del edge_attr, batch
    f32 = jnp.float32
    lyr = params['layers']

    embw, embb = params['node_emb']
    pew, peb = params['pe_lin']
    bw1, bb1 = params['budget_w1']
    bw2, bb2 = params['budget_w2']
    linw, linb = params['lin']

    pe_p = _pad_lanes(pe)
    pew_p = jnp.pad(pew, ((0, 128 - _PED), (0, 0)))
    src3 = edge_index[0].reshape(_NB, 1, _EPB)
    dst3 = edge_index[1].reshape(_NB, 1, _EPB)

    bw1_p = _pad_lanes(bw1)
    bb1_p = _pad_lanes(bb1[None, :])
    bw2_p = jnp.pad(bw2, ((0, _W - _BH), (0, _W - 2 * _L)))
    bb2_p = _pad_lanes(bb2[None, :])
    linw_p = _pad_lanes(linw)
    linb_p = _pad_lanes(linb[None, :])

    g1w = jnp.stack([lp['gin_w1'][0] for lp in lyr])
    g1b = jnp.stack([lp['gin_w1'][1] for lp in lyr])[:, None, :]
    g2w = jnp.stack([lp['gin_w2'][0] for lp in lyr])
    g2b = jnp.stack([lp['gin_w2'][1] for lp in lyr])[:, None, :]
    eps = jnp.stack([lp['eps'] for lp in lyr]).reshape(1, _L)
    sw = jnp.stack([lp['score'][0][:, 0] for lp in lyr])[:, None, :]

    def _heads_in(key):
        w = jnp.stack([lp[key][0] for lp in lyr]).reshape(_L, _C, _H, _DH)
        w = w.transpose(0, 2, 1, 3)
        b = jnp.stack([lp[key][1] for lp in lyr]).reshape(_L, _H, 1, _DH)
        return w, b

    wq, bq = _heads_in('wq')
    wk, bk = _heads_in('wk')
    wv, bv = _heads_in('wv')
    wo = jnp.stack([lp['wo'][0] for lp in lyr]).reshape(_L, _H, _DH, _C)
    bo = jnp.stack([lp['wo'][1] for lp in lyr])[:, None, :]

    operands = (
        x, pe_p, src3, dst3,
        embw, embb[None, :], pew_p, peb[None, :],
        bw1_p, bb1_p, bw2_p, bb2_p,
        g1w, g1b, g2w, g2b, eps,
        sw,
        wq, bq, wk, bk, wv, bv,
        wo, bo,
        linw_p, linb_p,
    )

    in_specs = [
        pl.BlockSpec((_R, _FEA), lambda i: (i, 0)),
        pl.BlockSpec((_R, 128), lambda i: (i, 0)),
        pl.BlockSpec((1, 1, _EPB), lambda i: (i, 0, 0)),
        pl.BlockSpec((1, 1, _EPB), lambda i: (i, 0, 0)),
    ] + [_full(op.shape) for op in operands[4:]]

    out_shape = (
        jax.ShapeDtypeStruct((_G, _W), f32),
        jax.ShapeDtypeStruct((_G, _W), f32),
        jax.ShapeDtypeStruct((_G, _W), f32),
    )
    out_specs = (
        pl.BlockSpec((_GPB, _W), lambda i: (i, 0)),
        pl.BlockSpec((_GPB, _W), lambda i: (i, 0)),
        pl.BlockSpec((_GPB, _W), lambda i: (i, 0)),
    )

    logits_p, tr_p, lg_p = pl.pallas_call(
        _fused_body,
        grid=(_NB,),
        in_specs=in_specs,
        out_specs=out_specs,
        out_shape=out_shape,
    )(*operands)

    logits = logits_p[:, :_T]
    tr = tr_p[:, :_L]
    lg = lg_p[:, _L:2 * _L]

    costs = [tr[:, l].mean() ** 2 * lg[:, l].mean() for l in range(_L)]
    dense_macs = float(_G * _H * _NPG * _NPG * _DH * 2 + 6 * _N * _C * _C)
    avg_compute = sum(costs) / float(_L)
    total_actual = sum(costs) * dense_macs
    total_dense = jnp.float32(dense_macs * _L)
    return (logits, avg_compute, tr, lg, total_actual, total_dense)


# speedup vs baseline: 2.3881x; 1.5662x over previous
import functools

import jax
import jax.numpy as jnp
import numpy as np
from jax import lax
from jax.experimental import pallas as pl

_G = 64
_NPG = 32
_N = _G * _NPG
_EPG = 512
_E = _G * _EPG
_FEA = 128
_C = 256
_L = 4
_T = 10
_H = 8
_DH = _C // _H
_PED = 20
_BH = 64
_MINR = 0.1

_NB = 16
_GPB = _G // _NB
_R = _GPB * _NPG
_EPB = _GPB * _EPG
_ECH = 1024
_W = 128

_dot = functools.partial(lax.dot_general, preferred_element_type=jnp.float32,
                         precision=lax.Precision.HIGHEST)
_dot_fast = functools.partial(lax.dot_general, preferred_element_type=jnp.float32)


def _fused_body(x_ref, pe_ref, src_ref, dst_ref,
                embw_ref, embb_ref, pew_ref, peb_ref,
                bw1_ref, bb1_ref, bw2_ref, bb2_ref,
                g1w_ref, g1b_ref, g2w_ref, g2b_ref, eps_ref,
                sw_ref,
                wq_ref, bq_ref, wk_ref, bk_ref, wv_ref, bv_ref,
                wo_ref, bo_ref,
                linw_ref, linb_ref,
                logits_ref, tr_ref, lg_ref):
    i = pl.program_id(0)
    f32 = jnp.float32

    row_id = lax.broadcasted_iota(jnp.int32, (_R, _R), 0)
    col_id = lax.broadcasted_iota(jnp.int32, (_R, _R), 1)
    same = (row_id // _NPG) == (col_id // _NPG)
    same_f = same.astype(f32)
    eye_f = (row_id == col_id).astype(f32)
    ones_rr = jnp.ones((_R, _R), f32)
    p_sum = ((lax.broadcasted_iota(jnp.int32, (_GPB, _R), 1) // _NPG)
             == lax.broadcasted_iota(jnp.int32, (_GPB, _R), 0)).astype(f32)
    lane_w = lax.broadcasted_iota(jnp.int32, (1, _W), 1)

    def _bcast_cols(col):
        return _dot(ones_rr, eye_f * col, (((1,), (0,)), ((), ())))

    h = _dot(x_ref[...], embw_ref[...], (((1,), (0,)), ((), ()))) + embb_ref[...]
    h = h + _dot(pe_ref[...], pew_ref[...], (((1,), (0,)), ((), ()))) + peb_ref[...]

    pooled = _dot(same_f, h, (((1,), (0,)), ((), ()))) * (1.0 / _NPG)
    hid = jnp.maximum(
        _dot(pooled, bw1_ref[...], (((1,), (0,)), ((), ()))) + bb1_ref[...], 0.0)
    bd = _dot(hid, bw2_ref[...], (((1,), (0,)), ((), ()))) + bb2_ref[...]
    trs = _MINR + (1.0 - _MINR) * jax.nn.sigmoid(bd)
    lgs = jax.nn.sigmoid(bd)
    tr_ref[...] = (_dot(p_sum, trs, (((1,), (0,)), ((), ()))) * (1.0 / _NPG))[None]
    lg_ref[...] = (_dot(p_sum, lgs, (((1,), (0,)), ((), ()))) * (1.0 / _NPG))[None]

    base = i * _R
    srcl = src_ref[0] - base
    dstl = dst_ref[0] - base
    rid = lax.broadcasted_iota(jnp.int32, (_R, _ECH), 0)
    adj = jnp.zeros((_R, _R), f32)
    for c in range(_EPB // _ECH):
        sl = srcl[:, c * _ECH:(c + 1) * _ECH]
        dl = dstl[:, c * _ECH:(c + 1) * _ECH]
        ohd = (dl == rid).astype(jnp.bfloat16)
        ohs = (sl == rid).astype(jnp.bfloat16)
        adj = adj + _dot_fast(ohd, ohs, (((1,), (1,)), ((), ())))

    neg = jnp.float32(-1e30)
    inv_sqrt = jnp.float32(1.0 / np.sqrt(_DH))

    for l in range(_L):
        agg = _dot(adj, h, (((1,), (0,)), ((), ())))
        eps = eps_ref[0:1, l:l + 1]
        z = h + eps * h + agg
        t1 = jnp.maximum(
            _dot(z, g1w_ref[l], (((1,), (0,)), ((), ()))) + g1b_ref[l], 0.0)
        h_local = h + _dot(t1, g2w_ref[l], (((1,), (0,)), ((), ()))) + g2b_ref[l]

        s_col = jnp.sum(h * sw_ref[l], axis=1, keepdims=True)
        smat = _bcast_cols(s_col)
        lt = smat < s_col
        eq = smat == s_col
        rank_col = jnp.sum(
            (same & (lt | (eq & (col_id < row_id)))).astype(f32),
            axis=1, keepdims=True)

        trl = jnp.sum(trs * (lane_w == l).astype(f32), axis=1, keepdims=True)
        lgl = jnp.sum(lgs * (lane_w == (_L + l)).astype(f32), axis=1, keepdims=True)
        idx_col = jnp.clip(jnp.floor((1.0 - trl) * float(_NPG - 1)),
                           0.0, float(_NPG - 1))
        dmat = _bcast_cols(rank_col - idx_col)
        selmat = (dmat == 0.0).astype(f32)
        thr_col = jnp.sum(same_f * smat * selmat, axis=1, keepdims=True)
        m_col = jax.nn.sigmoid(s_col - thr_col)

        bias = jnp.where(same, _bcast_cols(jnp.log(m_col + 1e-6)), neg)
        q = _dot(h, wq_ref[l], (((1,), (0,)), ((), ()))) + bq_ref[l]
        k = _dot(h, wk_ref[l], (((1,), (0,)), ((), ()))) + bk_ref[l]
        v = _dot(h, wv_ref[l], (((1,), (0,)), ((), ()))) + bv_ref[l]
        ovs = []
        for hh in range(_H):
            qh = q[:, hh * _DH:(hh + 1) * _DH]
            kh = k[:, hh * _DH:(hh + 1) * _DH]
            vh = v[:, hh * _DH:(hh + 1) * _DH]
            sc = _dot(qh, kh, (((1,), (1,)), ((), ()))) * inv_sqrt + bias
            p = jax.nn.softmax(sc, axis=-1)
            ovs.append(_dot(p, vh, (((1,), (0,)), ((), ()))))
        o_mat = jnp.concatenate(ovs, axis=1)
        o = (_dot(o_mat, wo_ref[l], (((1,), (0,)), ((), ()))) + bo_ref[l]) * m_col
        hsum = h_local + h + lgl * o
        mu = jnp.mean(hsum, axis=-1, keepdims=True)
        d = hsum - mu
        var = jnp.mean(d * d, axis=-1, keepdims=True)
        h = d / jnp.sqrt(var + 1e-5)

    out = _dot(h, linw_ref[...], (((1,), (0,)), ((), ())))
    logits_ref[...] = (_dot(p_sum, out, (((1,), (0,)), ((), ())))
                       + linb_ref[...])[None]


def _full(shape):
    nd = len(shape)
    return pl.BlockSpec(shape, lambda i, _nd=nd: (0,) * _nd)


def _pad_lanes(a, w=_W):
    return jnp.pad(a, ((0, 0),) * (a.ndim - 1) + ((0, w - a.shape[-1]),))


def kernel(x, pe, edge_index, edge_attr, batch, params):
    del edge_attr, batch
    f32 = jnp.float32
    lyr = params['layers']

    embw, embb = params['node_emb']
    pew, peb = params['pe_lin']
    bw1, bb1 = params['budget_w1']
    bw2, bb2 = params['budget_w2']
    linw, linb = params['lin']

    pe_p = _pad_lanes(pe)
    pew_p = jnp.pad(pew, ((0, 128 - _PED), (0, 0)))
    src3 = edge_index[0].reshape(_NB, 1, _EPB)
    dst3 = edge_index[1].reshape(_NB, 1, _EPB)

    bw1_p = _pad_lanes(bw1)
    bb1_p = _pad_lanes(bb1[None, :])
    bw2_p = jnp.pad(bw2, ((0, _W - _BH), (0, _W - 2 * _L)))
    bb2_p = _pad_lanes(bb2[None, :])
    linw_p = _pad_lanes(linw)
    linb_p = _pad_lanes(linb[None, :])

    g1w = jnp.stack([lp['gin_w1'][0] for lp in lyr])
    g1b = jnp.stack([lp['gin_w1'][1] for lp in lyr])[:, None, :]
    g2w = jnp.stack([lp['gin_w2'][0] for lp in lyr])
    g2b = jnp.stack([lp['gin_w2'][1] for lp in lyr])[:, None, :]
    eps = jnp.stack([lp['eps'] for lp in lyr]).reshape(1, _L)
    sw = jnp.stack([lp['score'][0][:, 0] for lp in lyr])[:, None, :]

    wq = jnp.stack([lp['wq'][0] for lp in lyr])
    bq = jnp.stack([lp['wq'][1] for lp in lyr])[:, None, :]
    wk = jnp.stack([lp['wk'][0] for lp in lyr])
    bk = jnp.stack([lp['wk'][1] for lp in lyr])[:, None, :]
    wv = jnp.stack([lp['wv'][0] for lp in lyr])
    bv = jnp.stack([lp['wv'][1] for lp in lyr])[:, None, :]
    wo = jnp.stack([lp['wo'][0] for lp in lyr])
    bo = jnp.stack([lp['wo'][1] for lp in lyr])[:, None, :]

    operands = (
        x, pe_p, src3, dst3,
        embw, embb[None, :], pew_p, peb[None, :],
        bw1_p, bb1_p, bw2_p, bb2_p,
        g1w, g1b, g2w, g2b, eps,
        sw,
        wq, bq, wk, bk, wv, bv,
        wo, bo,
        linw_p, linb_p,
    )

    in_specs = [
        pl.BlockSpec((_R, _FEA), lambda i: (i, 0)),
        pl.BlockSpec((_R, 128), lambda i: (i, 0)),
        pl.BlockSpec((1, 1, _EPB), lambda i: (i, 0, 0)),
        pl.BlockSpec((1, 1, _EPB), lambda i: (i, 0, 0)),
    ] + [_full(op.shape) for op in operands[4:]]

    out_shape = (
        jax.ShapeDtypeStruct((_NB, _GPB, _W), f32),
        jax.ShapeDtypeStruct((_NB, _GPB, _W), f32),
        jax.ShapeDtypeStruct((_NB, _GPB, _W), f32),
    )
    out_specs = (
        pl.BlockSpec((1, _GPB, _W), lambda i: (i, 0, 0)),
        pl.BlockSpec((1, _GPB, _W), lambda i: (i, 0, 0)),
        pl.BlockSpec((1, _GPB, _W), lambda i: (i, 0, 0)),
    )

    logits_p, tr_p, lg_p = pl.pallas_call(
        _fused_body,
        grid=(_NB,),
        in_specs=in_specs,
        out_specs=out_specs,
        out_shape=out_shape,
    )(*operands)

    logits = logits_p.reshape(_G, _W)[:, :_T]
    tr = tr_p.reshape(_G, _W)[:, :_L]
    lg = lg_p.reshape(_G, _W)[:, _L:2 * _L]

    costs = [tr[:, l].mean() ** 2 * lg[:, l].mean() for l in range(_L)]
    dense_macs = float(_G * _H * _NPG * _NPG * _DH * 2 + 6 * _N * _C * _C)
    avg_compute = sum(costs) / float(_L)
    total_actual = sum(costs) * dense_macs
    total_dense = jnp.float32(dense_macs * _L)
    return (logits, avg_compute, tr, lg, total_actual, total_dense)
